# baseline (speedup 1.0000x reference)
import jax
import jax.numpy as jnp
from jax.experimental import pallas as pl
from jax.experimental.pallas import tpu as pltpu

_N = 4096
_H = 64
_NCLS = 10
_IB = 512
_NI = _N // _IB


def _layer1_kernel(A_ref, x_ref, W1_ref, b1_ref, Abf_ref, h1_ref, B_scr):
    i = pl.program_id(0)

    @pl.when(i == 0)
    def _():
        B_scr[...] = jnp.dot(x_ref[...], W1_ref[...],
                             preferred_element_type=jnp.float32)

    Abf = A_ref[...].astype(jnp.bfloat16)
    Abf_ref[...] = Abf
    agg = jax.lax.dot_general(Abf, B_scr[...].astype(jnp.bfloat16),
                              (((0,), (0,)), ((), ())),
                              preferred_element_type=jnp.float32)
    h1_ref[...] = jnp.maximum(agg + b1_ref[...], 0.0)


def _layer23_kernel(Abf_ref, h1_ref, W2_ref, W3_ref, b2_ref, b3_ref,
                    LW1_ref, LW2_ref, LW3_ref, lb_ref,
                    out_ref, B_scr, h2_scr):
    p = pl.program_id(0)
    i = pl.program_id(1)

    @pl.when(jnp.logical_and(p == 0, i == 0))
    def _():
        B_scr[...] = jnp.dot(h1_ref[...], W2_ref[...],
                             preferred_element_type=jnp.float32)

    @pl.when(jnp.logical_and(p == 1, i == 0))
    def _():
        B_scr[...] = jnp.dot(h2_scr[...], W3_ref[...],
                             preferred_element_type=jnp.float32)

    agg = jax.lax.dot_general(Abf_ref[...], B_scr[...].astype(jnp.bfloat16),
                              (((0,), (0,)), ((), ())),
                              preferred_element_type=jnp.float32)

    @pl.when(p == 0)
    def _():
        h2_scr[pl.ds(i * _IB, _IB), :] = jnp.maximum(agg + b2_ref[...], 0.0)

    @pl.when(p == 1)
    def _():
        x3 = agg + b3_ref[...]
        x1 = h1_ref[pl.ds(i * _IB, _IB), :]
        x2 = h2_scr[pl.ds(i * _IB, _IB), :]
        logits = (jnp.dot(x1, LW1_ref[...], preferred_element_type=jnp.float32)
                  + jnp.dot(x2, LW2_ref[...], preferred_element_type=jnp.float32)
                  + jnp.dot(x3, LW3_ref[...], preferred_element_type=jnp.float32)
                  + lb_ref[...])
        m = jnp.max(logits, axis=1, keepdims=True)
        s = logits - m
        lse = jnp.log(jnp.sum(jnp.exp(s), axis=1, keepdims=True))
        out_ref[...] = s - lse


def kernel(x, edge_index, W1, W2, W3, b1, b2, b3, lin_W, lin_b):
    n, d_in = x.shape
    A = edge_index

    Abf, h1 = pl.pallas_call(
        _layer1_kernel,
        grid=(_NI,),
        in_specs=[
            pl.BlockSpec((_N, _IB), lambda i: (0, i)),
            pl.BlockSpec((_N, d_in), lambda i: (0, 0)),
            pl.BlockSpec((d_in, _H), lambda i: (0, 0)),
            pl.BlockSpec((1, _H), lambda i: (0, 0)),
        ],
        out_specs=[
            pl.BlockSpec((_N, _IB), lambda i: (0, i)),
            pl.BlockSpec((_IB, _H), lambda i: (i, 0)),
        ],
        out_shape=[
            jax.ShapeDtypeStruct((_N, _N), jnp.bfloat16),
            jax.ShapeDtypeStruct((_N, _H), jnp.float32),
        ],
        scratch_shapes=[pltpu.VMEM((_N, _H), jnp.float32)],
    )(A, x, W1, b1.reshape(1, _H))

    full = lambda r, c: pl.BlockSpec((r, c), lambda p, i: (0, 0))
    out = pl.pallas_call(
        _layer23_kernel,
        grid=(2, _NI),
        in_specs=[
            pl.BlockSpec((_N, _IB), lambda p, i: (0, i)),
            full(_N, _H),
            full(_H, _H), full(_H, _H),
            full(1, _H), full(1, _H),
            full(_H, _NCLS), full(_H, _NCLS), full(_H, _NCLS),
            full(1, _NCLS),
        ],
        out_specs=pl.BlockSpec((_IB, _NCLS), lambda p, i: (i, 0)),
        out_shape=jax.ShapeDtypeStruct((_N, _NCLS), jnp.float32),
        scratch_shapes=[
            pltpu.VMEM((_N, _H), jnp.float32),
            pltpu.VMEM((_N, _H), jnp.float32),
        ],
    )(
        Abf, h1, W2, W3,
        b2.reshape(1, _H), b3.reshape(1, _H),
        lin_W[:_H], lin_W[_H:2 * _H], lin_W[2 * _H:],
        lin_b.reshape(1, _NCLS),
    )
    return out


# speedup vs baseline: 1.0373x; 1.0373x over previous
import jax
import jax.numpy as jnp
from jax.experimental import pallas as pl
from jax.experimental.pallas import tpu as pltpu

_N = 4096
_H = 64
_NCLS = 10
_IB = 512
_NI = _N // _IB


def _layer1_kernel(A_ref, xt_ref, W1t_ref, b1_ref, Abf_ref, h1t_ref, Bt_scr):
    i = pl.program_id(0)

    @pl.when(i == 0)
    def _():
        Bt_scr[...] = jnp.dot(W1t_ref[...], xt_ref[...],
                              preferred_element_type=jnp.float32
                              ).astype(jnp.bfloat16)

    Abf = A_ref[...].astype(jnp.bfloat16)
    Abf_ref[...] = Abf
    agg_t = jnp.dot(Bt_scr[...], Abf,
                    preferred_element_type=jnp.float32)
    h1t_ref[...] = jnp.maximum(agg_t + b1_ref[...], 0.0)


def _layer23_kernel(Abf_ref, h1t_ref, W2t_ref, W3t_ref, b2_ref, b3_ref,
                    LW1t_ref, LW2t_ref, LW3t_ref, lb_ref,
                    out_ref, Bt_scr, h2t_scr):
    p = pl.program_id(0)
    i = pl.program_id(1)

    @pl.when(jnp.logical_and(p == 0, i == 0))
    def _():
        Bt_scr[...] = jnp.dot(W2t_ref[...], h1t_ref[...],
                              preferred_element_type=jnp.float32
                              ).astype(jnp.bfloat16)

    @pl.when(jnp.logical_and(p == 1, i == 0))
    def _():
        Bt_scr[...] = jnp.dot(W3t_ref[...], h2t_scr[...],
                              preferred_element_type=jnp.float32
                              ).astype(jnp.bfloat16)

    agg_t = jnp.dot(Bt_scr[...], Abf_ref[...],
                    preferred_element_type=jnp.float32)

    @pl.when(p == 0)
    def _():
        h2t_scr[:, pl.ds(i * _IB, _IB)] = jnp.maximum(agg_t + b2_ref[...], 0.0)

    @pl.when(p == 1)
    def _():
        x3t = agg_t + b3_ref[...]
        x1t = h1t_ref[:, pl.ds(i * _IB, _IB)]
        x2t = h2t_scr[:, pl.ds(i * _IB, _IB)]
        logits = (jnp.dot(LW1t_ref[...], x1t, preferred_element_type=jnp.float32)
                  + jnp.dot(LW2t_ref[...], x2t, preferred_element_type=jnp.float32)
                  + jnp.dot(LW3t_ref[...], x3t, preferred_element_type=jnp.float32)
                  + lb_ref[...])
        m = jnp.max(logits, axis=0, keepdims=True)
        s = logits - m
        lse = jnp.log(jnp.sum(jnp.exp(s), axis=0, keepdims=True))
        out_ref[...] = s - lse


def kernel(x, edge_index, W1, W2, W3, b1, b2, b3, lin_W, lin_b):
    n, d_in = x.shape
    A = edge_index

    Abf, h1t = pl.pallas_call(
        _layer1_kernel,
        grid=(_NI,),
        in_specs=[
            pl.BlockSpec((_N, _IB), lambda i: (0, i)),
            pl.BlockSpec((d_in, _N), lambda i: (0, 0)),
            pl.BlockSpec((_H, d_in), lambda i: (0, 0)),
            pl.BlockSpec((_H, 1), lambda i: (0, 0)),
        ],
        out_specs=[
            pl.BlockSpec((_N, _IB), lambda i: (0, i)),
            pl.BlockSpec((_H, _IB), lambda i: (0, i)),
        ],
        out_shape=[
            jax.ShapeDtypeStruct((_N, _N), jnp.bfloat16),
            jax.ShapeDtypeStruct((_H, _N), jnp.float32),
        ],
        scratch_shapes=[pltpu.VMEM((_H, _N), jnp.bfloat16)],
    )(A, x.T, W1.T, b1.reshape(_H, 1))

    full = lambda r, c: pl.BlockSpec((r, c), lambda p, i: (0, 0))
    out_t = pl.pallas_call(
        _layer23_kernel,
        grid=(2, _NI),
        in_specs=[
            pl.BlockSpec((_N, _IB), lambda p, i: (0, i)),
            full(_H, _N),
            full(_H, _H), full(_H, _H),
            full(_H, 1), full(_H, 1),
            full(_NCLS, _H), full(_NCLS, _H), full(_NCLS, _H),
            full(_NCLS, 1),
        ],
        out_specs=pl.BlockSpec((_NCLS, _IB), lambda p, i: (0, i)),
        out_shape=jax.ShapeDtypeStruct((_NCLS, _N), jnp.float32),
        scratch_shapes=[
            pltpu.VMEM((_H, _N), jnp.bfloat16),
            pltpu.VMEM((_H, _N), jnp.float32),
        ],
    )(
        Abf, h1t, W2.T, W3.T,
        b2.reshape(_H, 1), b3.reshape(_H, 1),
        lin_W[:_H].T, lin_W[_H:2 * _H].T, lin_W[2 * _H:].T,
        lin_b.reshape(_NCLS, 1),
    )
    return out_t.T


# speedup vs baseline: 1.0374x; 1.0001x over previous
import jax
import jax.numpy as jnp
from jax.experimental import pallas as pl
from jax.experimental.pallas import tpu as pltpu

_N = 4096
_H = 64
_NCLS = 10
_IB1 = 512
_NI1 = _N // _IB1
_IB2 = 1024
_NI2 = _N // _IB2


def _layer1_kernel(A_ref, x_ref, W1_ref, b1_ref, Abf_ref, h1t_ref, Bt_scr):
    i = pl.program_id(0)

    @pl.when(i == 0)
    def _():
        B = jnp.dot(x_ref[...], W1_ref[...],
                    preferred_element_type=jnp.float32)
        Bt_scr[...] = B.T.astype(jnp.bfloat16)

    Abf = A_ref[...].astype(jnp.bfloat16)
    Abf_ref[...] = Abf
    agg_t = jnp.dot(Bt_scr[...], Abf,
                    preferred_element_type=jnp.float32)
    h1t_ref[...] = jnp.maximum(agg_t + b1_ref[...], 0.0)


def _layer23_kernel(Abf_ref, h1t_ref, W2t_ref, W3t_ref, b2_ref, b3_ref,
                    LW1t_ref, LW2t_ref, LW3t_ref, lb_ref,
                    out_ref, Bt_scr, h2t_scr):
    p = pl.program_id(0)
    i = pl.program_id(1)

    @pl.when(jnp.logical_and(p == 0, i == 0))
    def _():
        Bt_scr[...] = jnp.dot(W2t_ref[...], h1t_ref[...],
                              preferred_element_type=jnp.float32
                              ).astype(jnp.bfloat16)

    @pl.when(jnp.logical_and(p == 1, i == 0))
    def _():
        Bt_scr[...] = jnp.dot(W3t_ref[...], h2t_scr[...],
                              preferred_element_type=jnp.float32
                              ).astype(jnp.bfloat16)

    agg_t = jnp.dot(Bt_scr[...], Abf_ref[...],
                    preferred_element_type=jnp.float32)

    @pl.when(p == 0)
    def _():
        h2t_scr[:, pl.ds(i * _IB2, _IB2)] = jnp.maximum(agg_t + b2_ref[...],
                                                        0.0)

    @pl.when(p == 1)
    def _():
        x3t = agg_t + b3_ref[...]
        x1t = h1t_ref[:, pl.ds(i * _IB2, _IB2)]
        x2t = h2t_scr[:, pl.ds(i * _IB2, _IB2)]
        logits = (jnp.dot(LW1t_ref[...], x1t, preferred_element_type=jnp.float32)
                  + jnp.dot(LW2t_ref[...], x2t, preferred_element_type=jnp.float32)
                  + jnp.dot(LW3t_ref[...], x3t, preferred_element_type=jnp.float32)
                  + lb_ref[...])
        m = jnp.max(logits, axis=0, keepdims=True)
        s = logits - m
        lse = jnp.log(jnp.sum(jnp.exp(s), axis=0, keepdims=True))
        out_ref[...] = (s - lse).T


def kernel(x, edge_index, W1, W2, W3, b1, b2, b3, lin_W, lin_b):
    n, d_in = x.shape
    A = edge_index

    Abf, h1t = pl.pallas_call(
        _layer1_kernel,
        grid=(_NI1,),
        in_specs=[
            pl.BlockSpec((_N, _IB1), lambda i: (0, i)),
            pl.BlockSpec((_N, d_in), lambda i: (0, 0)),
            pl.BlockSpec((d_in, _H), lambda i: (0, 0)),
            pl.BlockSpec((_H, 1), lambda i: (0, 0)),
        ],
        out_specs=[
            pl.BlockSpec((_N, _IB1), lambda i: (0, i)),
            pl.BlockSpec((_H, _IB1), lambda i: (0, i)),
        ],
        out_shape=[
            jax.ShapeDtypeStruct((_N, _N), jnp.bfloat16),
            jax.ShapeDtypeStruct((_H, _N), jnp.float32),
        ],
        scratch_shapes=[pltpu.VMEM((_H, _N), jnp.bfloat16)],
    )(A, x, W1, b1.reshape(_H, 1))

    full = lambda r, c: pl.BlockSpec((r, c), lambda p, i: (0, 0))
    out = pl.pallas_call(
        _layer23_kernel,
        grid=(2, _NI2),
        in_specs=[
            pl.BlockSpec((_N, _IB2), lambda p, i: (0, i)),
            full(_H, _N),
            full(_H, _H), full(_H, _H),
            full(_H, 1), full(_H, 1),
            full(_NCLS, _H), full(_NCLS, _H), full(_NCLS, _H),
            full(_NCLS, 1),
        ],
        out_specs=pl.BlockSpec((_IB2, _NCLS), lambda p, i: (i, 0)),
        out_shape=jax.ShapeDtypeStruct((_N, _NCLS), jnp.float32),
        scratch_shapes=[
            pltpu.VMEM((_H, _N), jnp.bfloat16),
            pltpu.VMEM((_H, _N), jnp.float32),
        ],
    )(
        Abf, h1t, W2.T, W3.T,
        b2.reshape(_H, 1), b3.reshape(_H, 1),
        lin_W[:_H].T, lin_W[_H:2 * _H].T, lin_W[2 * _H:].T,
        lin_b.reshape(_NCLS, 1),
    )
    return out


# speedup vs baseline: 1.1577x; 1.1160x over previous
import jax
import jax.numpy as jnp
from jax.experimental import pallas as pl
from jax.experimental.pallas import tpu as pltpu

_N = 4096
_H = 64
_NCLS = 10
_IB1 = 512
_NI1 = _N // _IB1
_IB2 = 1024
_NI2 = _N // _IB2

_TDIMS = (((0,), (0,)), ((), ()))


def _layer1_kernel(A_ref, x_ref, W1_ref, b1_ref, Abf_ref, h1t_ref, Bt_scr):
    i = pl.program_id(0)

    @pl.when(i == 0)
    def _():
        B = jnp.dot(x_ref[...], W1_ref[...],
                    preferred_element_type=jnp.float32)
        Bt_scr[...] = B.T.astype(jnp.bfloat16)

    Abf = A_ref[...].astype(jnp.bfloat16)
    Abf_ref[...] = Abf
    agg_t = jnp.dot(Bt_scr[...], Abf,
                    preferred_element_type=jnp.float32)
    h1t_ref[...] = jnp.maximum(agg_t + b1_ref[...].T, 0.0)


def _layer23_kernel(Abf_ref, h1t_ref, W2_ref, W3_ref, b2_ref, b3_ref,
                    LW_ref, lb_ref, out_ref, Bt_scr, h2t_scr):
    p = pl.program_id(0)
    i = pl.program_id(1)

    @pl.when(jnp.logical_and(p == 0, i == 0))
    def _():
        Bt_scr[...] = jax.lax.dot_general(
            W2_ref[...], h1t_ref[...], _TDIMS,
            preferred_element_type=jnp.float32).astype(jnp.bfloat16)

    @pl.when(jnp.logical_and(p == 1, i == 0))
    def _():
        Bt_scr[...] = jax.lax.dot_general(
            W3_ref[...], h2t_scr[...], _TDIMS,
            preferred_element_type=jnp.float32).astype(jnp.bfloat16)

    agg_t = jnp.dot(Bt_scr[...], Abf_ref[...],
                    preferred_element_type=jnp.float32)

    @pl.when(p == 0)
    def _():
        h2t_scr[:, pl.ds(i * _IB2, _IB2)] = jnp.maximum(
            agg_t + b2_ref[...].T, 0.0)

    @pl.when(p == 1)
    def _():
        x3t = agg_t + b3_ref[...].T
        x1t = h1t_ref[:, pl.ds(i * _IB2, _IB2)]
        x2t = h2t_scr[:, pl.ds(i * _IB2, _IB2)]
        LW = LW_ref[...]
        logits = (jax.lax.dot_general(LW[:_H], x1t, _TDIMS,
                                      preferred_element_type=jnp.float32)
                  + jax.lax.dot_general(LW[_H:2 * _H], x2t, _TDIMS,
                                        preferred_element_type=jnp.float32)
                  + jax.lax.dot_general(LW[2 * _H:], x3t, _TDIMS,
                                        preferred_element_type=jnp.float32)
                  + lb_ref[...].T)
        m = jnp.max(logits, axis=0, keepdims=True)
        s = logits - m
        lse = jnp.log(jnp.sum(jnp.exp(s), axis=0, keepdims=True))
        out_ref[...] = (s - lse).T


def kernel(x, edge_index, W1, W2, W3, b1, b2, b3, lin_W, lin_b):
    n, d_in = x.shape
    A = edge_index

    Abf, h1t = pl.pallas_call(
        _layer1_kernel,
        grid=(_NI1,),
        in_specs=[
            pl.BlockSpec((_N, _IB1), lambda i: (0, i)),
            pl.BlockSpec((_N, d_in), lambda i: (0, 0)),
            pl.BlockSpec((d_in, _H), lambda i: (0, 0)),
            pl.BlockSpec((1, _H), lambda i: (0, 0)),
        ],
        out_specs=[
            pl.BlockSpec((_N, _IB1), lambda i: (0, i)),
            pl.BlockSpec((_H, _IB1), lambda i: (0, i)),
        ],
        out_shape=[
            jax.ShapeDtypeStruct((_N, _N), jnp.bfloat16),
            jax.ShapeDtypeStruct((_H, _N), jnp.float32),
        ],
        scratch_shapes=[pltpu.VMEM((_H, _N), jnp.bfloat16)],
    )(A, x, W1, b1.reshape(1, _H))

    full = lambda r, c: pl.BlockSpec((r, c), lambda p, i: (0, 0))
    out = pl.pallas_call(
        _layer23_kernel,
        grid=(2, _NI2),
        in_specs=[
            pl.BlockSpec((_N, _IB2), lambda p, i: (0, i)),
            full(_H, _N),
            full(_H, _H), full(_H, _H),
            full(1, _H), full(1, _H),
            full(3 * _H, _NCLS),
            full(1, _NCLS),
        ],
        out_specs=pl.BlockSpec((_IB2, _NCLS), lambda p, i: (i, 0)),
        out_shape=jax.ShapeDtypeStruct((_N, _NCLS), jnp.float32),
        scratch_shapes=[
            pltpu.VMEM((_H, _N), jnp.bfloat16),
            pltpu.VMEM((_H, _N), jnp.float32),
        ],
    )(
        Abf, h1t, W2, W3,
        b2.reshape(1, _H), b3.reshape(1, _H),
        lin_W, lin_b.reshape(1, _NCLS),
    )
    return out


# speedup vs baseline: 1.1628x; 1.0044x over previous
import jax
import jax.numpy as jnp
from jax.experimental import pallas as pl
from jax.experimental.pallas import tpu as pltpu

_N = 4096
_H = 64
_NCLS = 10
_IB1 = 1024
_NI1 = _N // _IB1
_IB2 = 1024
_NI2 = _N // _IB2

_TDIMS = (((0,), (0,)), ((), ()))


def _layer1_kernel(A_ref, x_ref, W1_ref, b1_ref, Abf_ref, h1t_ref, Bt_scr):
    i = pl.program_id(0)

    @pl.when(i == 0)
    def _():
        B = jnp.dot(x_ref[...], W1_ref[...],
                    preferred_element_type=jnp.float32)
        Bt_scr[...] = B.T.astype(jnp.bfloat16)

    Abf = A_ref[...].astype(jnp.bfloat16)
    Abf_ref[...] = Abf
    agg_t = jnp.dot(Bt_scr[...], Abf,
                    preferred_element_type=jnp.float32)
    h1t_ref[...] = jnp.maximum(agg_t + b1_ref[...].T, 0.0)


def _layer23_kernel(Abf_ref, h1t_ref, W2_ref, W3_ref, b2_ref, b3_ref,
                    LW_ref, lb_ref, out_ref, Bt_scr, h2t_scr):
    p = pl.program_id(0)
    i = pl.program_id(1)

    @pl.when(jnp.logical_and(p == 0, i == 0))
    def _():
        Bt_scr[...] = jax.lax.dot_general(
            W2_ref[...], h1t_ref[...], _TDIMS,
            preferred_element_type=jnp.float32).astype(jnp.bfloat16)

    @pl.when(jnp.logical_and(p == 1, i == 0))
    def _():
        Bt_scr[...] = jax.lax.dot_general(
            W3_ref[...], h2t_scr[...], _TDIMS,
            preferred_element_type=jnp.float32).astype(jnp.bfloat16)

    agg_t = jnp.dot(Bt_scr[...], Abf_ref[...],
                    preferred_element_type=jnp.float32)

    @pl.when(p == 0)
    def _():
        h2t_scr[:, pl.ds(i * _IB2, _IB2)] = jnp.maximum(
            agg_t + b2_ref[...].T, 0.0)

    @pl.when(p == 1)
    def _():
        x3t = agg_t + b3_ref[...].T
        x1t = h1t_ref[:, pl.ds(i * _IB2, _IB2)]
        x2t = h2t_scr[:, pl.ds(i * _IB2, _IB2)]
        LW = LW_ref[...]
        logits = (jax.lax.dot_general(LW[:_H], x1t, _TDIMS,
                                      preferred_element_type=jnp.float32)
                  + jax.lax.dot_general(LW[_H:2 * _H], x2t, _TDIMS,
                                        preferred_element_type=jnp.float32)
                  + jax.lax.dot_general(LW[2 * _H:], x3t, _TDIMS,
                                        preferred_element_type=jnp.float32)
                  + lb_ref[...].T)
        m = jnp.max(logits, axis=0, keepdims=True)
        s = logits - m
        lse = jnp.log(jnp.sum(jnp.exp(s), axis=0, keepdims=True))
        out_ref[...] = (s - lse).T


def kernel(x, edge_index, W1, W2, W3, b1, b2, b3, lin_W, lin_b):
    n, d_in = x.shape
    A = edge_index

    Abf, h1t = pl.pallas_call(
        _layer1_kernel,
        grid=(_NI1,),
        in_specs=[
            pl.BlockSpec((_N, _IB1), lambda i: (0, i)),
            pl.BlockSpec((_N, d_in), lambda i: (0, 0)),
            pl.BlockSpec((d_in, _H), lambda i: (0, 0)),
            pl.BlockSpec((1, _H), lambda i: (0, 0)),
        ],
        out_specs=[
            pl.BlockSpec((_N, _IB1), lambda i: (0, i)),
            pl.BlockSpec((_H, _IB1), lambda i: (0, i)),
        ],
        out_shape=[
            jax.ShapeDtypeStruct((_N, _N), jnp.bfloat16),
            jax.ShapeDtypeStruct((_H, _N), jnp.float32),
        ],
        scratch_shapes=[pltpu.VMEM((_H, _N), jnp.bfloat16)],
    )(A, x, W1, b1.reshape(1, _H))

    full = lambda r, c: pl.BlockSpec((r, c), lambda p, i: (0, 0))
    out = pl.pallas_call(
        _layer23_kernel,
        grid=(2, _NI2),
        in_specs=[
            pl.BlockSpec((_N, _IB2), lambda p, i: (0, i)),
            full(_H, _N),
            full(_H, _H), full(_H, _H),
            full(1, _H), full(1, _H),
            full(3 * _H, _NCLS),
            full(1, _NCLS),
        ],
        out_specs=pl.BlockSpec((_IB2, _NCLS), lambda p, i: (i, 0)),
        out_shape=jax.ShapeDtypeStruct((_N, _NCLS), jnp.float32),
        scratch_shapes=[
            pltpu.VMEM((_H, _N), jnp.bfloat16),
            pltpu.VMEM((_H, _N), jnp.float32),
        ],
    )(
        Abf, h1t, W2, W3,
        b2.reshape(1, _H), b3.reshape(1, _H),
        lin_W, lin_b.reshape(1, _NCLS),
    )
    return out


# speedup vs baseline: 1.5278x; 1.3139x over previous
import jax
import jax.numpy as jnp
from jax.experimental import pallas as pl
from jax.experimental.pallas import tpu as pltpu

_N = 4096
_H = 64
_NCLS = 10
_IB = 512
_NI = _N // _IB

_TDIMS = (((0,), (0,)), ((), ()))


def _gcn_kernel(A_ref, x_ref, W1_ref, W2_ref, W3_ref,
                b1_ref, b2_ref, b3_ref, LW_ref, lb_ref,
                out_ref, Abig_scr, Bt_scr, h1t_scr, h2t_scr):
    p = pl.program_id(0)
    i = pl.program_id(1)

    @pl.when(jnp.logical_and(p == 0, i == 0))
    def _():
        B = jnp.dot(x_ref[...], W1_ref[...],
                    preferred_element_type=jnp.float32)
        Bt_scr[...] = B.T.astype(jnp.bfloat16)

    @pl.when(jnp.logical_and(p == 1, i == 0))
    def _():
        Bt_scr[...] = jax.lax.dot_general(
            W2_ref[...], h1t_scr[...], _TDIMS,
            preferred_element_type=jnp.float32).astype(jnp.bfloat16)

    @pl.when(jnp.logical_and(p == 2, i == 0))
    def _():
        Bt_scr[...] = jax.lax.dot_general(
            W3_ref[...], h2t_scr[...], _TDIMS,
            preferred_element_type=jnp.float32).astype(jnp.bfloat16)

    cols = pl.ds(i * _IB, _IB)

    @pl.when(p == 0)
    def _():
        Abf = A_ref[...].astype(jnp.bfloat16)
        Abig_scr[:, cols] = Abf
        agg_t = jnp.dot(Bt_scr[...], Abf,
                        preferred_element_type=jnp.float32)
        h1t_scr[:, cols] = jnp.maximum(agg_t + b1_ref[...].T, 0.0)

    @pl.when(p == 1)
    def _():
        agg_t = jnp.dot(Bt_scr[...], Abig_scr[:, cols],
                        preferred_element_type=jnp.float32)
        h2t_scr[:, cols] = jnp.maximum(agg_t + b2_ref[...].T, 0.0)

    @pl.when(p == 2)
    def _():
        agg_t = jnp.dot(Bt_scr[...], Abig_scr[:, cols],
                        preferred_element_type=jnp.float32)
        x3t = agg_t + b3_ref[...].T
        x1t = h1t_scr[:, cols]
        x2t = h2t_scr[:, cols]
        LW = LW_ref[...]
        logits = (jax.lax.dot_general(LW[:_H], x1t, _TDIMS,
                                      preferred_element_type=jnp.float32)
                  + jax.lax.dot_general(LW[_H:2 * _H], x2t, _TDIMS,
                                        preferred_element_type=jnp.float32)
                  + jax.lax.dot_general(LW[2 * _H:], x3t, _TDIMS,
                                        preferred_element_type=jnp.float32)
                  + lb_ref[...].T)
        m = jnp.max(logits, axis=0, keepdims=True)
        s = logits - m
        lse = jnp.log(jnp.sum(jnp.exp(s), axis=0, keepdims=True))
        out_ref[...] = (s - lse).T


def kernel(x, edge_index, W1, W2, W3, b1, b2, b3, lin_W, lin_b):
    n, d_in = x.shape
    A = edge_index

    full = lambda r, c: pl.BlockSpec((r, c), lambda p, i: (0, 0))
    out = pl.pallas_call(
        _gcn_kernel,
        grid=(3, _NI),
        in_specs=[
            pl.BlockSpec((_N, _IB), lambda p, i: (0, i * (p == 0))),
            full(_N, d_in),
            full(d_in, _H), full(_H, _H), full(_H, _H),
            full(1, _H), full(1, _H), full(1, _H),
            full(3 * _H, _NCLS),
            full(1, _NCLS),
        ],
        out_specs=pl.BlockSpec((_IB, _NCLS), lambda p, i: (i, 0)),
        out_shape=jax.ShapeDtypeStruct((_N, _NCLS), jnp.float32),
        scratch_shapes=[
            pltpu.VMEM((_N, _N), jnp.bfloat16),
            pltpu.VMEM((_H, _N), jnp.bfloat16),
            pltpu.VMEM((_H, _N), jnp.float32),
            pltpu.VMEM((_H, _N), jnp.float32),
        ],
    )(
        A, x, W1, W2, W3,
        b1.reshape(1, _H), b2.reshape(1, _H), b3.reshape(1, _H),
        lin_W, lin_b.reshape(1, _NCLS),
    )
    return out
